# baseline (speedup 1.0000x reference)
import functools

import jax
import jax.numpy as jnp
import numpy as np
from jax import lax
from jax.experimental import pallas as pl
from jax.experimental.pallas import tpu as pltpu
from jax.experimental.pallas import tpu_sc as plsc

D_MODEL = 768
N_HEADS = 12
HEAD_DIM = 64
HALF = 32
D_LATENT = 384
E = 8
KSEL = 2
FF = 1536
EPS = 1e-5
ROPE_BASE = 10000.0
SEQ = 2048
CAP = 512
NSLOT = E * CAP
SBLK = 256
FFC = 384


def _proj_body(x_ref, n1_ref, wq_ref, wdkv_ref, wuk_ref, wuv_ref,
               cos_ref, sins_ref, q_ref, k_ref, v_ref):
    xb = x_ref[...]
    ms = jnp.mean(xb * xb, axis=-1, keepdims=True)
    xn = (xb * lax.rsqrt(ms + EPS) * n1_ref[...]).astype(jnp.bfloat16)
    q = jnp.dot(xn, wq_ref[...].astype(jnp.bfloat16),
                preferred_element_type=jnp.float32)
    latb = jnp.dot(xn, wdkv_ref[...].astype(jnp.bfloat16),
                   preferred_element_type=jnp.float32).astype(jnp.bfloat16)
    k = jnp.dot(latb, wuk_ref[...].astype(jnp.bfloat16),
                preferred_element_type=jnp.float32)
    v = jnp.dot(latb, wuv_ref[...].astype(jnp.bfloat16),
                preferred_element_type=jnp.float32)
    cos = cos_ref[...]
    sins = sins_ref[...]
    fh = (lax.broadcasted_iota(jnp.int32, (SBLK, D_MODEL), 1) % HEAD_DIM) < HALF

    def rope(t):
        shift = jnp.where(fh, jnp.roll(t, -HALF, axis=1), jnp.roll(t, HALF, axis=1))
        return (t * cos + shift * sins).astype(jnp.bfloat16)

    q_ref[...] = rope(q)
    k_ref[...] = rope(k)
    v_ref[...] = v.astype(jnp.bfloat16)


def _attn_body(q_ref, k_ref, v_ref, x_ref, wo_ref, n2_ref, wr_ref,
               x2_ref, xn2_ref, lg_ref, s_ref, av_ref):
    i = pl.program_id(0)
    nblk = SEQ // SBLK
    scale = 1.0 / np.sqrt(HEAD_DIM)
    row = i * SBLK + lax.broadcasted_iota(jnp.int32, (SBLK, SEQ), 0)
    col = lax.broadcasted_iota(jnp.int32, (SBLK, SEQ), 1)
    causal = col <= row
    aos = []
    for h in range(N_HEADS):
        hs = slice(h * HEAD_DIM, (h + 1) * HEAD_DIM)
        qh = q_ref[:, hs]
        for kb in range(nblk):
            @pl.when(kb <= i)
            def _(kb=kb, qh=qh, hs=hs):
                kc = k_ref[kb * SBLK:(kb + 1) * SBLK, hs]
                s_ref[:, kb * SBLK:(kb + 1) * SBLK] = lax.dot_general(
                    qh, kc, (((1,), (1,)), ((), ())),
                    preferred_element_type=jnp.float32) * scale
        s = jnp.where(causal, s_ref[...], -1e9)
        m = jnp.max(s, axis=-1, keepdims=True)
        p = jnp.exp(s - m)
        pb = (p / jnp.sum(p, axis=-1, keepdims=True)).astype(jnp.bfloat16)
        av_ref[...] = jnp.zeros((SBLK, HEAD_DIM), jnp.float32)
        for kb in range(nblk):
            @pl.when(kb <= i)
            def _(kb=kb, pb=pb, hs=hs):
                vc = v_ref[kb * SBLK:(kb + 1) * SBLK, hs]
                av_ref[...] += jnp.dot(pb[:, kb * SBLK:(kb + 1) * SBLK], vc,
                                       preferred_element_type=jnp.float32)
        aos.append(av_ref[...])
    ao = jnp.concatenate(aos, axis=-1).astype(jnp.bfloat16)
    wo = wo_ref[...].astype(jnp.bfloat16)
    x2 = x_ref[...] + jnp.dot(ao, wo, preferred_element_type=jnp.float32)
    ms = jnp.mean(x2 * x2, axis=-1, keepdims=True)
    xn2 = x2 * lax.rsqrt(ms + EPS) * n2_ref[...]
    x2_ref[...] = x2
    xn2_ref[...] = xn2
    lg_ref[...] = jnp.dot(xn2, wr_ref[...], preferred_element_type=jnp.float32,
                          precision=lax.Precision.HIGHEST)


def _route_body(lg_ref, src_ref, sw_ref, info_ref):
    lg = lg_ref[...]
    m = jnp.max(lg, axis=-1, keepdims=True)
    pe = jnp.exp(lg - m)
    p = pe / jnp.sum(pe, axis=-1, keepdims=True)
    iota8 = lax.broadcasted_iota(jnp.int32, (SEQ, E), 1)
    t1 = jnp.max(p, axis=-1, keepdims=True)
    i1 = jnp.min(jnp.where(p == t1, iota8, E), axis=-1, keepdims=True)
    oh1 = iota8 == i1
    p2 = jnp.where(oh1, -1.0, p)
    t2 = jnp.max(p2, axis=-1, keepdims=True)
    i2 = jnp.min(jnp.where(p2 == t2, iota8, E), axis=-1, keepdims=True)
    oh2 = iota8 == i2
    gsum = t1 + t2
    w1 = t1 / gsum
    w2 = t2 / gsum
    oh1f = oh1.astype(jnp.float32)
    oh2f = oh2.astype(jnp.float32)
    cnt = oh1f + oh2f
    tri = (lax.broadcasted_iota(jnp.int32, (128, 128), 0)
           >= lax.broadcasted_iota(jnp.int32, (128, 128), 1)).astype(jnp.float32)
    chunks = []
    carry = jnp.zeros((1, E), jnp.float32)
    for i in range(SEQ // 128):
        blk = cnt[i * 128:(i + 1) * 128, :]
        chunks.append(jnp.dot(tri, blk, preferred_element_type=jnp.float32) + carry)
        carry = carry + jnp.sum(blk, axis=0, keepdims=True)
    cum = jnp.concatenate(chunks, axis=0)
    excl = cum - cnt
    pos1 = jnp.sum(jnp.where(oh1, excl, 0.0), axis=-1, keepdims=True)
    pos2 = jnp.sum(jnp.where(oh2, excl, 0.0), axis=-1, keepdims=True)
    keep1 = pos1 < CAP
    keep2 = pos2 < CAP
    posc1 = jnp.minimum(pos1, CAP - 1.0)
    posc2 = jnp.minimum(pos2, CAP - 1.0)
    iota_c = lax.broadcasted_iota(jnp.int32, (SEQ, CAP), 1).astype(jnp.float32)
    a1 = (pos1 == iota_c).astype(jnp.float32)
    a2 = (pos2 == iota_c).astype(jnp.float32)
    toki = lax.broadcasted_iota(jnp.int32, (SEQ, 1), 0)
    th = (toki // 16).astype(jnp.float32)
    tl = (toki % 16).astype(jnp.float32)
    dn = (((0,), (0,)), ((), ()))

    def dd(a, b):
        return lax.dot_general(a, b, dn, preferred_element_type=jnp.float32)

    src = (dd(th * oh1f, a1) + dd(th * oh2f, a2)) * 16.0 \
        + dd(tl * oh1f, a1) + dd(tl * oh2f, a2)
    sw = dd(a1, w1 * oh1f) + dd(a2, w2 * oh2f)
    n_e = jnp.minimum(carry, float(CAP))
    nmin = jnp.min(n_e, axis=-1, keepdims=True)
    iota8r = lax.broadcasted_iota(jnp.int32, (1, E), 1).astype(jnp.float32)
    ez = jnp.min(jnp.where(n_e == nmin, iota8r, float(E)), axis=-1, keepdims=True)
    dz = jnp.minimum(ez * CAP + nmin, float(NSLOT - 1))
    i1f = i1.astype(jnp.float32)
    i2f = i2.astype(jnp.float32)
    d1 = jnp.where(keep1, i1f * CAP + posc1, dz)
    d2 = jnp.where(keep2, i2f * CAP + posc2, dz)
    info = jnp.concatenate([d1, d2, d1, d1, d1, d1, d1, d1], axis=1)
    src_ref[...] = src.astype(jnp.int32)
    sw_ref[...] = sw
    info_ref[...] = info.astype(jnp.int32)


def _ffn_body(buf_ref, wg_ref, wu_ref, wd_ref, sw_ref, ob_ref):
    f = pl.program_id(1)
    e = pl.program_id(0)
    b = buf_ref[...].astype(jnp.bfloat16)
    hg = jnp.dot(b, wg_ref[0].astype(jnp.bfloat16),
                 preferred_element_type=jnp.float32)
    hu = jnp.dot(b, wu_ref[0].astype(jnp.bfloat16),
                 preferred_element_type=jnp.float32)
    act = hg * (1.0 / (1.0 + jnp.exp(-hg)))
    oc = jnp.dot((act * hu).astype(jnp.bfloat16), wd_ref[0].astype(jnp.bfloat16),
                 preferred_element_type=jnp.float32)

    @pl.when(f == 0)
    def _():
        ob_ref[...] = oc

    @pl.when(f > 0)
    def _():
        ob_ref[...] = ob_ref[...] + oc

    @pl.when(f == FF // FFC - 1)
    def _():
        swf = sw_ref[...]
        sel = lax.broadcasted_iota(jnp.int32, (CAP, E), 1) == e
        swcol = jnp.sum(jnp.where(sel, swf, 0.0), axis=-1, keepdims=True)
        ob_ref[...] = ob_ref[...] * swcol


def _sc_dispatch(src_all, idx_all):
    mesh = plsc.VectorSubcoreMesh(core_axis_name="c", subcore_axis_name="s")
    nper = NSLOT // 32
    dt = src_all.dtype

    @functools.partial(
        pl.kernel,
        out_type=jax.ShapeDtypeStruct((NSLOT, D_MODEL), dt),
        mesh=mesh,
        scratch_types=[pltpu.VMEM((nper,), jnp.int32),
                       pltpu.VMEM((nper, D_MODEL), dt),
                       pltpu.SemaphoreType.DMA],
    )
    def disp(src_hbm, idx_hbm, buf_hbm, idx_v, rows_v, sem):
        wid = lax.axis_index("s") * 2 + lax.axis_index("c")
        base = wid * nper
        pltpu.sync_copy(idx_hbm.at[pl.ds(base, nper)], idx_v)
        pltpu.async_copy(src_hbm.at[idx_v], rows_v, sem).wait()
        pltpu.sync_copy(rows_v, buf_hbm.at[pl.ds(base, nper)])

    return disp(src_all, idx_all)


def _sc_combine(sob, x2, d1, d2):
    mesh = plsc.VectorSubcoreMesh(core_axis_name="c", subcore_axis_name="s")
    tper = SEQ // 32
    half = tper // 2

    @functools.partial(
        pl.kernel,
        out_type=jax.ShapeDtypeStruct((SEQ, D_MODEL), jnp.float32),
        mesh=mesh,
        scratch_types=[pltpu.VMEM((half,), jnp.int32),
                       pltpu.VMEM((half,), jnp.int32),
                       pltpu.VMEM((half, D_MODEL), jnp.float32),
                       pltpu.VMEM((half, D_MODEL), jnp.float32),
                       pltpu.VMEM((half, D_MODEL), jnp.float32),
                       pltpu.SemaphoreType.DMA],
    )
    def comb(sob_hbm, x2_hbm, d1_hbm, d2_hbm, y_hbm,
             i1v, i2v, r1v, r2v, accv, sem):
        wid = lax.axis_index("s") * 2 + lax.axis_index("c")
        for pno in range(2):
            base = wid * tper + pno * half
            pltpu.sync_copy(d1_hbm.at[pl.ds(base, half)], i1v)
            pltpu.sync_copy(d2_hbm.at[pl.ds(base, half)], i2v)
            cp1 = pltpu.async_copy(sob_hbm.at[i1v], r1v, sem)
            cp2 = pltpu.async_copy(sob_hbm.at[i2v], r2v, sem)
            pltpu.sync_copy(x2_hbm.at[pl.ds(base, half)], accv)
            cp1.wait()
            cp2.wait()

            @plsc.parallel_loop(0, half, 1, unroll=2)
            def _(r):
                for cc in range(D_MODEL // 16):
                    c = cc * 16
                    accv[r, pl.ds(c, 16)] = (accv[r, pl.ds(c, 16)]
                                             + r1v[r, pl.ds(c, 16)]
                                             + r2v[r, pl.ds(c, 16)])

            pltpu.sync_copy(accv, y_hbm.at[pl.ds(base, half)])

    return comb(sob, x2, d1, d2)


def kernel(x, norm1_w, Wq, Wdkv, Wuk, Wuv, Wo, norm2_w, Wr, W_gate, W_up, W_down):
    xf = x.reshape(SEQ, D_MODEL)
    n1 = norm1_w.reshape(1, D_MODEL)
    n2 = norm2_w.reshape(1, D_MODEL)

    inv = 1.0 / (ROPE_BASE ** (jnp.arange(HALF, dtype=jnp.float32) / HALF))
    ang = jnp.arange(SEQ, dtype=jnp.float32)[:, None] * inv[None, :]
    cos = jnp.tile(jnp.cos(ang), (1, D_MODEL // HALF))
    sin32 = jnp.tile(jnp.sin(ang), (1, D_MODEL // HALF))
    sgn = jnp.where((jnp.arange(D_MODEL) % HEAD_DIM) < HALF, -1.0, 1.0)
    sins = sin32 * sgn[None, :]

    nblk = SEQ // SBLK
    row_spec = pl.BlockSpec((SBLK, D_MODEL), lambda i: (i, 0))
    full = lambda *s: pl.BlockSpec(s, lambda i: tuple(0 for _ in s))

    q, k, v = pl.pallas_call(
        _proj_body,
        grid=(nblk,),
        in_specs=[row_spec, full(1, D_MODEL), full(D_MODEL, D_MODEL),
                  full(D_MODEL, D_LATENT), full(D_LATENT, D_MODEL),
                  full(D_LATENT, D_MODEL), row_spec, row_spec],
        out_specs=[row_spec, row_spec, row_spec],
        out_shape=[jax.ShapeDtypeStruct((SEQ, D_MODEL), jnp.bfloat16)] * 3,
    )(xf, n1, Wq, Wdkv, Wuk, Wuv, cos, sins)

    x2, xn2, logits = pl.pallas_call(
        _attn_body,
        grid=(nblk,),
        in_specs=[row_spec, full(SEQ, D_MODEL), full(SEQ, D_MODEL),
                  row_spec, full(D_MODEL, D_MODEL),
                  full(1, D_MODEL), full(D_MODEL, E)],
        out_specs=[row_spec, row_spec,
                   pl.BlockSpec((SBLK, E), lambda i: (i, 0))],
        out_shape=[jax.ShapeDtypeStruct((SEQ, D_MODEL), jnp.float32),
                   jax.ShapeDtypeStruct((SEQ, D_MODEL), jnp.float32),
                   jax.ShapeDtypeStruct((SEQ, E), jnp.float32)],
        scratch_shapes=[pltpu.VMEM((SBLK, SEQ), jnp.float32),
                        pltpu.VMEM((SBLK, HEAD_DIM), jnp.float32)],
    )(q, k, v, xf, Wo, n2, Wr)

    slot_src, slot_w, info = pl.pallas_call(
        _route_body,
        out_shape=[jax.ShapeDtypeStruct((E, CAP), jnp.int32),
                   jax.ShapeDtypeStruct((CAP, E), jnp.float32),
                   jax.ShapeDtypeStruct((SEQ, E), jnp.int32)],
    )(logits)

    buf = _sc_dispatch(xn2, slot_src.reshape(NSLOT))

    sob = pl.pallas_call(
        _ffn_body,
        grid=(E, FF // FFC),
        in_specs=[pl.BlockSpec((CAP, D_MODEL), lambda e, f: (e, 0)),
                  pl.BlockSpec((1, D_MODEL, FFC), lambda e, f: (e, 0, f)),
                  pl.BlockSpec((1, D_MODEL, FFC), lambda e, f: (e, 0, f)),
                  pl.BlockSpec((1, FFC, D_MODEL), lambda e, f: (e, f, 0)),
                  pl.BlockSpec((CAP, E), lambda e, f: (0, 0))],
        out_specs=pl.BlockSpec((CAP, D_MODEL), lambda e, f: (e, 0)),
        out_shape=jax.ShapeDtypeStruct((NSLOT, D_MODEL), jnp.float32),
    )(buf, W_gate, W_up, W_down, slot_w)

    y = _sc_combine(sob, x2, info[:, 0], info[:, 1])
    return y.reshape(SEQ, 1, D_MODEL)


# speedup vs baseline: 1.2929x; 1.2929x over previous
import functools

import jax
import jax.numpy as jnp
import numpy as np
from jax import lax
from jax.experimental import pallas as pl
from jax.experimental.pallas import tpu as pltpu
from jax.experimental.pallas import tpu_sc as plsc

D_MODEL = 768
N_HEADS = 12
HEAD_DIM = 64
HALF = 32
D_LATENT = 384
E = 8
KSEL = 2
FF = 1536
EPS = 1e-5
ROPE_BASE = 10000.0
SEQ = 2048
CAP = 512
NSLOT = E * CAP
SBLK = 256
FFC = 384


def _proj_body(x_ref, n1_ref, wq_ref, wdkv_ref, wuk_ref, wuv_ref,
               cos_ref, sins_ref, q_ref, k_ref, v_ref):
    xb = x_ref[...]
    ms = jnp.mean(xb * xb, axis=-1, keepdims=True)
    xn = (xb * lax.rsqrt(ms + EPS) * n1_ref[...]).astype(jnp.bfloat16)
    q = jnp.dot(xn, wq_ref[...].astype(jnp.bfloat16),
                preferred_element_type=jnp.float32)
    latb = jnp.dot(xn, wdkv_ref[...].astype(jnp.bfloat16),
                   preferred_element_type=jnp.float32).astype(jnp.bfloat16)
    k = jnp.dot(latb, wuk_ref[...].astype(jnp.bfloat16),
                preferred_element_type=jnp.float32)
    v = jnp.dot(latb, wuv_ref[...].astype(jnp.bfloat16),
                preferred_element_type=jnp.float32)
    cos = cos_ref[...]
    sins = sins_ref[...]
    fh = (lax.broadcasted_iota(jnp.int32, (SBLK, D_MODEL), 1) % HEAD_DIM) < HALF

    def rope(t):
        shift = jnp.where(fh, jnp.roll(t, -HALF, axis=1), jnp.roll(t, HALF, axis=1))
        return (t * cos + shift * sins).astype(jnp.bfloat16)

    q_ref[...] = rope(q)
    k_ref[...] = rope(k)
    v_ref[...] = v.astype(jnp.bfloat16)


def _attn_body(q_ref, k_ref, v_ref, x_ref, wo_ref, n2_ref, wr_ref,
               x2_ref, xn2_ref, lg_ref):
    i = pl.program_id(0)
    scale = 1.0 / np.sqrt(HEAD_DIM)
    row = i * SBLK + lax.broadcasted_iota(jnp.int32, (SBLK, SEQ), 0)
    col = lax.broadcasted_iota(jnp.int32, (SBLK, SEQ), 1)
    causal = col <= row
    aos = []
    for h in range(N_HEADS):
        qh = q_ref[:, h * HEAD_DIM:(h + 1) * HEAD_DIM]
        kh = k_ref[:, h * HEAD_DIM:(h + 1) * HEAD_DIM]
        vh = v_ref[:, h * HEAD_DIM:(h + 1) * HEAD_DIM]
        s = lax.dot_general(qh, kh, (((1,), (1,)), ((), ())),
                            preferred_element_type=jnp.float32) * scale
        s = jnp.where(causal, s, -1e9)
        m = jnp.max(s, axis=-1, keepdims=True)
        p = jnp.exp(s - m)
        p = p / jnp.sum(p, axis=-1, keepdims=True)
        aos.append(jnp.dot(p.astype(jnp.bfloat16), vh,
                           preferred_element_type=jnp.float32))
    ao = jnp.concatenate(aos, axis=-1).astype(jnp.bfloat16)
    wo = wo_ref[...].astype(jnp.bfloat16)
    x2 = x_ref[...] + jnp.dot(ao, wo, preferred_element_type=jnp.float32)
    ms = jnp.mean(x2 * x2, axis=-1, keepdims=True)
    xn2 = x2 * lax.rsqrt(ms + EPS) * n2_ref[...]
    x2_ref[...] = x2
    xn2_ref[...] = xn2
    lg_ref[...] = jnp.dot(xn2, wr_ref[...], preferred_element_type=jnp.float32,
                          precision=lax.Precision.HIGHEST)


def _route_body(lg_ref, src_ref, sw_ref, info_ref):
    lg = lg_ref[...]
    m = jnp.max(lg, axis=-1, keepdims=True)
    pe = jnp.exp(lg - m)
    p = pe / jnp.sum(pe, axis=-1, keepdims=True)
    iota8 = lax.broadcasted_iota(jnp.int32, (SEQ, E), 1)
    t1 = jnp.max(p, axis=-1, keepdims=True)
    i1 = jnp.min(jnp.where(p == t1, iota8, E), axis=-1, keepdims=True)
    oh1 = iota8 == i1
    p2 = jnp.where(oh1, -1.0, p)
    t2 = jnp.max(p2, axis=-1, keepdims=True)
    i2 = jnp.min(jnp.where(p2 == t2, iota8, E), axis=-1, keepdims=True)
    oh2 = iota8 == i2
    gsum = t1 + t2
    w1 = t1 / gsum
    w2 = t2 / gsum
    oh1f = oh1.astype(jnp.float32)
    oh2f = oh2.astype(jnp.float32)
    cnt = oh1f + oh2f
    tri = (lax.broadcasted_iota(jnp.int32, (128, 128), 0)
           >= lax.broadcasted_iota(jnp.int32, (128, 128), 1)).astype(jnp.float32)
    chunks = []
    carry = jnp.zeros((1, E), jnp.float32)
    for i in range(SEQ // 128):
        blk = cnt[i * 128:(i + 1) * 128, :]
        chunks.append(jnp.dot(tri, blk, preferred_element_type=jnp.float32) + carry)
        carry = carry + jnp.sum(blk, axis=0, keepdims=True)
    cum = jnp.concatenate(chunks, axis=0)
    excl = cum - cnt
    pos1 = jnp.sum(jnp.where(oh1, excl, 0.0), axis=-1, keepdims=True)
    pos2 = jnp.sum(jnp.where(oh2, excl, 0.0), axis=-1, keepdims=True)
    keep1 = pos1 < CAP
    keep2 = pos2 < CAP
    posc1 = jnp.minimum(pos1, CAP - 1.0)
    posc2 = jnp.minimum(pos2, CAP - 1.0)
    iota_c = lax.broadcasted_iota(jnp.int32, (SEQ, CAP), 1).astype(jnp.float32)
    a1 = (pos1 == iota_c).astype(jnp.float32)
    a2 = (pos2 == iota_c).astype(jnp.float32)
    toki = lax.broadcasted_iota(jnp.int32, (SEQ, 1), 0)
    th = (toki // 16).astype(jnp.float32)
    tl = (toki % 16).astype(jnp.float32)
    dn = (((0,), (0,)), ((), ()))

    def dd(a, b):
        return lax.dot_general(a, b, dn, preferred_element_type=jnp.float32)

    src = (dd(th * oh1f, a1) + dd(th * oh2f, a2)) * 16.0 \
        + dd(tl * oh1f, a1) + dd(tl * oh2f, a2)
    sw = dd(a1, w1 * oh1f) + dd(a2, w2 * oh2f)
    n_e = jnp.minimum(carry, float(CAP))
    nmin = jnp.min(n_e, axis=-1, keepdims=True)
    iota8r = lax.broadcasted_iota(jnp.int32, (1, E), 1).astype(jnp.float32)
    ez = jnp.min(jnp.where(n_e == nmin, iota8r, float(E)), axis=-1, keepdims=True)
    dz = jnp.minimum(ez * CAP + nmin, float(NSLOT - 1))
    i1f = i1.astype(jnp.float32)
    i2f = i2.astype(jnp.float32)
    d1 = jnp.where(keep1, i1f * CAP + posc1, dz)
    d2 = jnp.where(keep2, i2f * CAP + posc2, dz)
    info = jnp.concatenate([d1, d2, d1, d1, d1, d1, d1, d1], axis=1)
    src_ref[...] = src.astype(jnp.int32)
    sw_ref[...] = sw
    info_ref[...] = info.astype(jnp.int32)


def _ffn_body(buf_ref, wg_ref, wu_ref, wd_ref, sw_ref, ob_ref):
    f = pl.program_id(1)
    e = pl.program_id(0)
    b = buf_ref[...].astype(jnp.bfloat16)
    hg = jnp.dot(b, wg_ref[0], preferred_element_type=jnp.float32)
    hu = jnp.dot(b, wu_ref[0], preferred_element_type=jnp.float32)
    act = hg * (1.0 / (1.0 + jnp.exp(-hg)))
    oc = jnp.dot((act * hu).astype(jnp.bfloat16), wd_ref[0],
                 preferred_element_type=jnp.float32)

    @pl.when(f == 0)
    def _():
        ob_ref[...] = oc

    @pl.when(f > 0)
    def _():
        ob_ref[...] = ob_ref[...] + oc

    @pl.when(f == FF // FFC - 1)
    def _():
        swf = sw_ref[...]
        sel = lax.broadcasted_iota(jnp.int32, (CAP, E), 1) == e
        swcol = jnp.sum(jnp.where(sel, swf, 0.0), axis=-1, keepdims=True)
        ob_ref[...] = ob_ref[...] * swcol


def _sc_dispatch(src_all, idx_all):
    mesh = plsc.VectorSubcoreMesh(core_axis_name="c", subcore_axis_name="s")
    nper = NSLOT // 32
    dt = src_all.dtype

    @functools.partial(
        pl.kernel,
        out_type=jax.ShapeDtypeStruct((NSLOT, D_MODEL), dt),
        mesh=mesh,
        scratch_types=[pltpu.VMEM((nper,), jnp.int32),
                       pltpu.VMEM((nper, D_MODEL), dt),
                       pltpu.SemaphoreType.DMA],
    )
    def disp(src_hbm, idx_hbm, buf_hbm, idx_v, rows_v, sem):
        wid = lax.axis_index("s") * 2 + lax.axis_index("c")
        base = wid * nper
        pltpu.sync_copy(idx_hbm.at[pl.ds(base, nper)], idx_v)
        pltpu.async_copy(src_hbm.at[idx_v], rows_v, sem).wait()
        pltpu.sync_copy(rows_v, buf_hbm.at[pl.ds(base, nper)])

    return disp(src_all, idx_all)


def _sc_combine(sob, x2, d1, d2):
    mesh = plsc.VectorSubcoreMesh(core_axis_name="c", subcore_axis_name="s")
    tper = SEQ // 32
    half = tper // 2

    @functools.partial(
        pl.kernel,
        out_type=jax.ShapeDtypeStruct((SEQ, D_MODEL), jnp.float32),
        mesh=mesh,
        scratch_types=[pltpu.VMEM((half,), jnp.int32),
                       pltpu.VMEM((half,), jnp.int32),
                       pltpu.VMEM((half, D_MODEL), jnp.float32),
                       pltpu.VMEM((half, D_MODEL), jnp.float32),
                       pltpu.VMEM((half, D_MODEL), jnp.float32),
                       pltpu.SemaphoreType.DMA],
    )
    def comb(sob_hbm, x2_hbm, d1_hbm, d2_hbm, y_hbm,
             i1v, i2v, r1v, r2v, accv, sem):
        wid = lax.axis_index("s") * 2 + lax.axis_index("c")
        for pno in range(2):
            base = wid * tper + pno * half
            pltpu.sync_copy(d1_hbm.at[pl.ds(base, half)], i1v)
            pltpu.sync_copy(d2_hbm.at[pl.ds(base, half)], i2v)
            cp1 = pltpu.async_copy(sob_hbm.at[i1v], r1v, sem)
            cp2 = pltpu.async_copy(sob_hbm.at[i2v], r2v, sem)
            pltpu.sync_copy(x2_hbm.at[pl.ds(base, half)], accv)
            cp1.wait()
            cp2.wait()

            @plsc.parallel_loop(0, half, 1, unroll=2)
            def _(r):
                for cc in range(D_MODEL // 16):
                    c = cc * 16
                    accv[r, pl.ds(c, 16)] = (accv[r, pl.ds(c, 16)]
                                             + r1v[r, pl.ds(c, 16)]
                                             + r2v[r, pl.ds(c, 16)])

            pltpu.sync_copy(accv, y_hbm.at[pl.ds(base, half)])

    return comb(sob, x2, d1, d2)


def kernel(x, norm1_w, Wq, Wdkv, Wuk, Wuv, Wo, norm2_w, Wr, W_gate, W_up, W_down):
    xf = x.reshape(SEQ, D_MODEL)
    n1 = norm1_w.reshape(1, D_MODEL)
    n2 = norm2_w.reshape(1, D_MODEL)

    inv = 1.0 / (ROPE_BASE ** (jnp.arange(HALF, dtype=jnp.float32) / HALF))
    ang = jnp.arange(SEQ, dtype=jnp.float32)[:, None] * inv[None, :]
    cos = jnp.tile(jnp.cos(ang), (1, D_MODEL // HALF))
    sin32 = jnp.tile(jnp.sin(ang), (1, D_MODEL // HALF))
    sgn = jnp.where((jnp.arange(D_MODEL) % HEAD_DIM) < HALF, -1.0, 1.0)
    sins = sin32 * sgn[None, :]

    nblk = SEQ // SBLK
    row_spec = pl.BlockSpec((SBLK, D_MODEL), lambda i: (i, 0))
    full = lambda *s: pl.BlockSpec(s, lambda i: tuple(0 for _ in s))

    q, k, v = pl.pallas_call(
        _proj_body,
        grid=(nblk,),
        in_specs=[row_spec, full(1, D_MODEL), full(D_MODEL, D_MODEL),
                  full(D_MODEL, D_LATENT), full(D_LATENT, D_MODEL),
                  full(D_LATENT, D_MODEL), row_spec, row_spec],
        out_specs=[row_spec, row_spec, row_spec],
        out_shape=[jax.ShapeDtypeStruct((SEQ, D_MODEL), jnp.bfloat16)] * 3,
    )(xf, n1, Wq, Wdkv, Wuk, Wuv, cos, sins)

    x2, xn2, logits = pl.pallas_call(
        _attn_body,
        grid=(nblk,),
        in_specs=[row_spec, full(SEQ, D_MODEL), full(SEQ, D_MODEL),
                  row_spec, full(D_MODEL, D_MODEL),
                  full(1, D_MODEL), full(D_MODEL, E)],
        out_specs=[row_spec, row_spec,
                   pl.BlockSpec((SBLK, E), lambda i: (i, 0))],
        out_shape=[jax.ShapeDtypeStruct((SEQ, D_MODEL), jnp.float32),
                   jax.ShapeDtypeStruct((SEQ, D_MODEL), jnp.float32),
                   jax.ShapeDtypeStruct((SEQ, E), jnp.float32)],
    )(q, k, v, xf, Wo, n2, Wr)

    slot_src, slot_w, info = pl.pallas_call(
        _route_body,
        out_shape=[jax.ShapeDtypeStruct((E, CAP), jnp.int32),
                   jax.ShapeDtypeStruct((CAP, E), jnp.float32),
                   jax.ShapeDtypeStruct((SEQ, E), jnp.int32)],
    )(logits)

    buf = _sc_dispatch(xn2, slot_src.reshape(NSLOT))

    sob = pl.pallas_call(
        _ffn_body,
        grid=(E, FF // FFC),
        in_specs=[pl.BlockSpec((CAP, D_MODEL), lambda e, f: (e, 0)),
                  pl.BlockSpec((1, D_MODEL, FFC), lambda e, f: (e, 0, f)),
                  pl.BlockSpec((1, D_MODEL, FFC), lambda e, f: (e, 0, f)),
                  pl.BlockSpec((1, FFC, D_MODEL), lambda e, f: (e, f, 0)),
                  pl.BlockSpec((CAP, E), lambda e, f: (0, 0))],
        out_specs=pl.BlockSpec((CAP, D_MODEL), lambda e, f: (e, 0)),
        out_shape=jax.ShapeDtypeStruct((NSLOT, D_MODEL), jnp.float32),
    )(buf, W_gate.astype(jnp.bfloat16), W_up.astype(jnp.bfloat16),
      W_down.astype(jnp.bfloat16), slot_w)

    y = _sc_combine(sob, x2, info[:, 0], info[:, 1])
    return y.reshape(SEQ, 1, D_MODEL)


# speedup vs baseline: 1.4706x; 1.1374x over previous
import functools

import jax
import jax.numpy as jnp
import numpy as np
from jax import lax
from jax.experimental import pallas as pl
from jax.experimental.pallas import tpu as pltpu
from jax.experimental.pallas import tpu_sc as plsc

D_MODEL = 768
N_HEADS = 12
HEAD_DIM = 64
HALF = 32
D_LATENT = 384
E = 8
KSEL = 2
FF = 1536
EPS = 1e-5
ROPE_BASE = 10000.0
SEQ = 2048
CAP = 512
NSLOT = E * CAP
SBLK = 256
FFC = 384


def _proj_body(x_ref, n1_ref, wq_ref, wdkv_ref, wuk_ref, wuv_ref,
               cos_ref, sins_ref, q_ref, k_ref, v_ref):
    xb = x_ref[...]
    ms = jnp.mean(xb * xb, axis=-1, keepdims=True)
    xn = (xb * lax.rsqrt(ms + EPS) * n1_ref[...]).astype(jnp.bfloat16)
    q = jnp.dot(xn, wq_ref[...].astype(jnp.bfloat16),
                preferred_element_type=jnp.float32)
    latb = jnp.dot(xn, wdkv_ref[...].astype(jnp.bfloat16),
                   preferred_element_type=jnp.float32).astype(jnp.bfloat16)
    k = jnp.dot(latb, wuk_ref[...].astype(jnp.bfloat16),
                preferred_element_type=jnp.float32)
    v = jnp.dot(latb, wuv_ref[...].astype(jnp.bfloat16),
                preferred_element_type=jnp.float32)
    cos = cos_ref[...]
    sins = sins_ref[...]
    fh = (lax.broadcasted_iota(jnp.int32, (SBLK, D_MODEL), 1) % HEAD_DIM) < HALF

    def rope(t):
        shift = jnp.where(fh, jnp.roll(t, -HALF, axis=1), jnp.roll(t, HALF, axis=1))
        return (t * cos + shift * sins).astype(jnp.bfloat16)

    q_ref[...] = rope(q)
    k_ref[...] = rope(k)
    v_ref[...] = v.astype(jnp.bfloat16)


def _attn_body(q_ref, k_ref, v_ref, x_ref, wo_ref, n2_ref, wr_ref,
               x2_ref, xn2_ref, lg_ref):
    i = pl.program_id(0)
    scale = 1.0 / np.sqrt(HEAD_DIM)
    row = i * SBLK + lax.broadcasted_iota(jnp.int32, (SBLK, SEQ), 0)
    col = lax.broadcasted_iota(jnp.int32, (SBLK, SEQ), 1)
    causal = col <= row
    aos = []
    for h in range(N_HEADS):
        qh = q_ref[:, h * HEAD_DIM:(h + 1) * HEAD_DIM]
        kh = k_ref[:, h * HEAD_DIM:(h + 1) * HEAD_DIM]
        vh = v_ref[:, h * HEAD_DIM:(h + 1) * HEAD_DIM]
        s = lax.dot_general(qh, kh, (((1,), (1,)), ((), ())),
                            preferred_element_type=jnp.float32) * scale
        s = jnp.where(causal, s, -1e9)
        m = jnp.max(s, axis=-1, keepdims=True)
        p = jnp.exp(s - m)
        p = p / jnp.sum(p, axis=-1, keepdims=True)
        aos.append(jnp.dot(p.astype(jnp.bfloat16), vh,
                           preferred_element_type=jnp.float32))
    ao = jnp.concatenate(aos, axis=-1).astype(jnp.bfloat16)
    wo = wo_ref[...].astype(jnp.bfloat16)
    x2 = x_ref[...] + jnp.dot(ao, wo, preferred_element_type=jnp.float32)
    ms = jnp.mean(x2 * x2, axis=-1, keepdims=True)
    xn2 = x2 * lax.rsqrt(ms + EPS) * n2_ref[...]
    x2_ref[...] = x2
    xn2_ref[...] = xn2
    lg_ref[...] = jnp.dot(xn2, wr_ref[...], preferred_element_type=jnp.float32,
                          precision=lax.Precision.HIGHEST)


def _route_body(lg_ref, src_ref, sw_ref, info_ref):
    lg = lg_ref[...]
    m = jnp.max(lg, axis=-1, keepdims=True)
    pe = jnp.exp(lg - m)
    p = pe / jnp.sum(pe, axis=-1, keepdims=True)
    iota8 = lax.broadcasted_iota(jnp.int32, (SEQ, E), 1)
    t1 = jnp.max(p, axis=-1, keepdims=True)
    i1 = jnp.min(jnp.where(p == t1, iota8, E), axis=-1, keepdims=True)
    oh1 = iota8 == i1
    p2 = jnp.where(oh1, -1.0, p)
    t2 = jnp.max(p2, axis=-1, keepdims=True)
    i2 = jnp.min(jnp.where(p2 == t2, iota8, E), axis=-1, keepdims=True)
    oh2 = iota8 == i2
    gsum = t1 + t2
    w1 = t1 / gsum
    w2 = t2 / gsum
    oh1f = oh1.astype(jnp.float32)
    oh2f = oh2.astype(jnp.float32)
    cnt = oh1f + oh2f
    tri = (lax.broadcasted_iota(jnp.int32, (128, 128), 0)
           >= lax.broadcasted_iota(jnp.int32, (128, 128), 1)).astype(jnp.float32)
    chunks = []
    carry = jnp.zeros((1, E), jnp.float32)
    for i in range(SEQ // 128):
        blk = cnt[i * 128:(i + 1) * 128, :]
        chunks.append(jnp.dot(tri, blk, preferred_element_type=jnp.float32) + carry)
        carry = carry + jnp.sum(blk, axis=0, keepdims=True)
    cum = jnp.concatenate(chunks, axis=0)
    excl = cum - cnt
    pos1 = jnp.sum(jnp.where(oh1, excl, 0.0), axis=-1, keepdims=True)
    pos2 = jnp.sum(jnp.where(oh2, excl, 0.0), axis=-1, keepdims=True)
    keep1 = pos1 < CAP
    keep2 = pos2 < CAP
    posc1 = jnp.minimum(pos1, CAP - 1.0)
    posc2 = jnp.minimum(pos2, CAP - 1.0)
    iota_c = lax.broadcasted_iota(jnp.int32, (SEQ, CAP), 1).astype(jnp.float32)
    a1 = (pos1 == iota_c).astype(jnp.float32)
    a2 = (pos2 == iota_c).astype(jnp.float32)
    toki = lax.broadcasted_iota(jnp.int32, (SEQ, 1), 0)
    th = (toki // 16).astype(jnp.float32)
    tl = (toki % 16).astype(jnp.float32)
    dn = (((0,), (0,)), ((), ()))

    def dd(a, b):
        return lax.dot_general(a, b, dn, preferred_element_type=jnp.float32)

    src = (dd(th * oh1f, a1) + dd(th * oh2f, a2)) * 16.0 \
        + dd(tl * oh1f, a1) + dd(tl * oh2f, a2)
    sw = dd(a1, w1 * oh1f) + dd(a2, w2 * oh2f)
    n_e = jnp.minimum(carry, float(CAP))
    nmin = jnp.min(n_e, axis=-1, keepdims=True)
    iota8r = lax.broadcasted_iota(jnp.int32, (1, E), 1).astype(jnp.float32)
    ez = jnp.min(jnp.where(n_e == nmin, iota8r, float(E)), axis=-1, keepdims=True)
    dz = jnp.minimum(ez * CAP + nmin, float(NSLOT - 1))
    i1f = i1.astype(jnp.float32)
    i2f = i2.astype(jnp.float32)
    d1 = jnp.where(keep1, i1f * CAP + posc1, dz)
    d2 = jnp.where(keep2, i2f * CAP + posc2, dz)
    info = jnp.concatenate([d1, d2, d1, d1, d1, d1, d1, d1], axis=1)
    src_ref[...] = src.astype(jnp.int32)
    sw_ref[...] = sw
    info_ref[...] = info.astype(jnp.int32)


def _ffn_body(buf_ref, wg_ref, wu_ref, wd_ref, sw_ref, ob_ref):
    f = pl.program_id(1)
    e = pl.program_id(0)
    b = buf_ref[...].astype(jnp.bfloat16)
    hg = jnp.dot(b, wg_ref[0].astype(jnp.bfloat16),
                 preferred_element_type=jnp.float32)
    hu = jnp.dot(b, wu_ref[0].astype(jnp.bfloat16),
                 preferred_element_type=jnp.float32)
    act = hg * (1.0 / (1.0 + jnp.exp(-hg)))
    oc = jnp.dot((act * hu).astype(jnp.bfloat16), wd_ref[0].astype(jnp.bfloat16),
                 preferred_element_type=jnp.float32)

    @pl.when(f == 0)
    def _():
        ob_ref[...] = oc

    @pl.when(f > 0)
    def _():
        ob_ref[...] = ob_ref[...] + oc

    @pl.when(f == FF // FFC - 1)
    def _():
        swf = sw_ref[...]
        sel = lax.broadcasted_iota(jnp.int32, (CAP, E), 1) == e
        swcol = jnp.sum(jnp.where(sel, swf, 0.0), axis=-1, keepdims=True)
        ob_ref[...] = ob_ref[...] * swcol


def _sc_dispatch(src_all, idx_all):
    mesh = plsc.VectorSubcoreMesh(core_axis_name="c", subcore_axis_name="s")
    nper = NSLOT // 32
    dt = src_all.dtype

    @functools.partial(
        pl.kernel,
        out_type=jax.ShapeDtypeStruct((NSLOT, D_MODEL), dt),
        mesh=mesh,
        scratch_types=[pltpu.VMEM((nper,), jnp.int32),
                       pltpu.VMEM((nper, D_MODEL), dt),
                       pltpu.SemaphoreType.DMA],
    )
    def disp(src_hbm, idx_hbm, buf_hbm, idx_v, rows_v, sem):
        wid = lax.axis_index("s") * 2 + lax.axis_index("c")
        base = wid * nper
        pltpu.sync_copy(idx_hbm.at[pl.ds(base, nper)], idx_v)
        pltpu.async_copy(src_hbm.at[idx_v], rows_v, sem).wait()
        pltpu.sync_copy(rows_v, buf_hbm.at[pl.ds(base, nper)])

    return disp(src_all, idx_all)


def _sc_combine(sob, x2, d1, d2):
    mesh = plsc.VectorSubcoreMesh(core_axis_name="c", subcore_axis_name="s")
    tper = SEQ // 32
    half = tper // 2

    @functools.partial(
        pl.kernel,
        out_type=jax.ShapeDtypeStruct((SEQ, D_MODEL), jnp.float32),
        mesh=mesh,
        scratch_types=[pltpu.VMEM((half,), jnp.int32),
                       pltpu.VMEM((half,), jnp.int32),
                       pltpu.VMEM((half, D_MODEL), jnp.float32),
                       pltpu.VMEM((half, D_MODEL), jnp.float32),
                       pltpu.VMEM((half, D_MODEL), jnp.float32),
                       pltpu.SemaphoreType.DMA],
    )
    def comb(sob_hbm, x2_hbm, d1_hbm, d2_hbm, y_hbm,
             i1v, i2v, r1v, r2v, accv, sem):
        wid = lax.axis_index("s") * 2 + lax.axis_index("c")
        for pno in range(2):
            base = wid * tper + pno * half
            pltpu.sync_copy(d1_hbm.at[pl.ds(base, half)], i1v)
            pltpu.sync_copy(d2_hbm.at[pl.ds(base, half)], i2v)
            cp1 = pltpu.async_copy(sob_hbm.at[i1v], r1v, sem)
            cp2 = pltpu.async_copy(sob_hbm.at[i2v], r2v, sem)
            pltpu.sync_copy(x2_hbm.at[pl.ds(base, half)], accv)
            cp1.wait()
            cp2.wait()

            @plsc.parallel_loop(0, half, 1, unroll=2)
            def _(r):
                for cc in range(D_MODEL // 16):
                    c = cc * 16
                    accv[r, pl.ds(c, 16)] = (accv[r, pl.ds(c, 16)]
                                             + r1v[r, pl.ds(c, 16)]
                                             + r2v[r, pl.ds(c, 16)])

            pltpu.sync_copy(accv, y_hbm.at[pl.ds(base, half)])

    return comb(sob, x2, d1, d2)


def kernel(x, norm1_w, Wq, Wdkv, Wuk, Wuv, Wo, norm2_w, Wr, W_gate, W_up, W_down):
    xf = x.reshape(SEQ, D_MODEL)
    n1 = norm1_w.reshape(1, D_MODEL)
    n2 = norm2_w.reshape(1, D_MODEL)

    inv = 1.0 / (ROPE_BASE ** (jnp.arange(HALF, dtype=jnp.float32) / HALF))
    ang = jnp.arange(SEQ, dtype=jnp.float32)[:, None] * inv[None, :]
    cos = jnp.tile(jnp.cos(ang), (1, D_MODEL // HALF))
    sin32 = jnp.tile(jnp.sin(ang), (1, D_MODEL // HALF))
    sgn = jnp.where((jnp.arange(D_MODEL) % HEAD_DIM) < HALF, -1.0, 1.0)
    sins = sin32 * sgn[None, :]

    nblk = SEQ // SBLK
    row_spec = pl.BlockSpec((SBLK, D_MODEL), lambda i: (i, 0))
    full = lambda *s: pl.BlockSpec(s, lambda i: tuple(0 for _ in s))

    q, k, v = pl.pallas_call(
        _proj_body,
        grid=(nblk,),
        in_specs=[row_spec, full(1, D_MODEL), full(D_MODEL, D_MODEL),
                  full(D_MODEL, D_LATENT), full(D_LATENT, D_MODEL),
                  full(D_LATENT, D_MODEL), row_spec, row_spec],
        out_specs=[row_spec, row_spec, row_spec],
        out_shape=[jax.ShapeDtypeStruct((SEQ, D_MODEL), jnp.bfloat16)] * 3,
    )(xf, n1, Wq, Wdkv, Wuk, Wuv, cos, sins)

    x2, xn2, logits = pl.pallas_call(
        _attn_body,
        grid=(nblk,),
        in_specs=[row_spec, full(SEQ, D_MODEL), full(SEQ, D_MODEL),
                  row_spec, full(D_MODEL, D_MODEL),
                  full(1, D_MODEL), full(D_MODEL, E)],
        out_specs=[row_spec, row_spec,
                   pl.BlockSpec((SBLK, E), lambda i: (i, 0))],
        out_shape=[jax.ShapeDtypeStruct((SEQ, D_MODEL), jnp.float32),
                   jax.ShapeDtypeStruct((SEQ, D_MODEL), jnp.float32),
                   jax.ShapeDtypeStruct((SEQ, E), jnp.float32)],
    )(q, k, v, xf, Wo, n2, Wr)

    slot_src, slot_w, info = pl.pallas_call(
        _route_body,
        out_shape=[jax.ShapeDtypeStruct((E, CAP), jnp.int32),
                   jax.ShapeDtypeStruct((CAP, E), jnp.float32),
                   jax.ShapeDtypeStruct((SEQ, E), jnp.int32)],
    )(logits)

    buf = _sc_dispatch(xn2, slot_src.reshape(NSLOT))

    sob = pl.pallas_call(
        _ffn_body,
        grid=(E, FF // FFC),
        in_specs=[pl.BlockSpec((CAP, D_MODEL), lambda e, f: (e, 0)),
                  pl.BlockSpec((1, D_MODEL, FFC), lambda e, f: (e, 0, f)),
                  pl.BlockSpec((1, D_MODEL, FFC), lambda e, f: (e, 0, f)),
                  pl.BlockSpec((1, FFC, D_MODEL), lambda e, f: (e, f, 0)),
                  pl.BlockSpec((CAP, E), lambda e, f: (0, 0))],
        out_specs=pl.BlockSpec((CAP, D_MODEL), lambda e, f: (e, 0)),
        out_shape=jax.ShapeDtypeStruct((NSLOT, D_MODEL), jnp.float32),
    )(buf, W_gate, W_up, W_down, slot_w)

    y = _sc_combine(sob, x2, info[:, 0], info[:, 1])
    return y.reshape(SEQ, 1, D_MODEL)
